# baseline (speedup 1.0000x reference)
import functools

import jax
import jax.numpy as jnp
from jax import lax
from jax.experimental import pallas as pl
from jax.experimental.pallas import tpu as pltpu
from jax.experimental.pallas import tpu_sc as plsc

N_TOK = 8192
K = 8192
D = 256
BT = 1024
N_BLOCKS = N_TOK // BT

_NC = 2
_NS = 16
_NW = _NC * _NS
_B_PER_W = N_TOK // _NW


def _argmin_body(x_ref, c_ref, zn_ref, cn_ref, near_ref, perp_ref, cnt_ref):
    i = pl.program_id(0)

    @pl.when(i == 0)
    def _init():
        cnt_ref[...] = jnp.zeros_like(cnt_ref)

    x = x_ref[...]
    xb = (2.0 * x).astype(jnp.bfloat16)
    dn = (((1,), (1,)), ((), ()))
    Q = K // 4
    kio = lax.broadcasted_iota(jnp.int32, (BT, Q), 1)
    zn = zn_ref[...]

    def quarter(q):
        conv = lax.dot_general(xb, c_ref[q * Q:(q + 1) * Q], dn,
                               preferred_element_type=jnp.float32)
        return (zn - conv) + cn_ref[:, q * Q:(q + 1) * Q]

    dq = [quarter(q) for q in range(4)]
    mq = [jnp.min(d, axis=1, keepdims=True) for d in dq]
    m0 = jnp.minimum(mq[0], mq[1])
    m1 = jnp.minimum(mq[2], mq[3])
    iq = [jnp.min(jnp.where(dq[q] == (m0 if q < 2 else m1), kio + q * Q, K),
                  axis=1, keepdims=True) for q in range(4)]
    i0 = jnp.minimum(iq[0], iq[1])
    i1 = jnp.minimum(iq[2], iq[3])
    a = m0.astype(jnp.bfloat16).astype(jnp.float32)
    idx = jnp.where(a <= m1, i0, i1)
    near_ref[...] = idx

    hi = idx >> 7
    lo = idx & 127
    oh_hi = (hi == lax.broadcasted_iota(jnp.int32, (BT, 64), 1)
             ).astype(jnp.float32)
    oh_lo = (lo == lax.broadcasted_iota(jnp.int32, (BT, 128), 1)
             ).astype(jnp.float32)
    cnt_ref[...] += lax.dot_general(oh_hi, oh_lo, (((0,), (0,)), ((), ())),
                                    preferred_element_type=jnp.float32)

    @pl.when(i == N_BLOCKS - 1)
    def _fin():
        e = cnt_ref[...] * (1.0 / N_TOK)
        ent = jnp.sum(e * jnp.log(e + 1e-10))
        perp_ref[...] = jnp.broadcast_to(jnp.exp(-ent), (1, 1))


def _nearest_and_perplexity(x_flat, codebook, znorm, cnorm):
    return pl.pallas_call(
        _argmin_body,
        grid=(N_BLOCKS,),
        in_specs=[
            pl.BlockSpec((BT, D), lambda i: (i, 0)),
            pl.BlockSpec((K, D), lambda i: (0, 0)),
            pl.BlockSpec((BT, 1), lambda i: (i, 0)),
            pl.BlockSpec((1, K), lambda i: (0, 0)),
        ],
        out_specs=[
            pl.BlockSpec((BT, 1), lambda i: (i, 0)),
            pl.BlockSpec((1, 1), lambda i: (0, 0)),
        ],
        out_shape=[
            jax.ShapeDtypeStruct((N_TOK, 1), jnp.int32),
            jax.ShapeDtypeStruct((1, 1), jnp.float32),
        ],
        scratch_shapes=[
            pltpu.VMEM((64, 128), jnp.float32),
        ],
    )(x_flat, codebook, znorm, cnorm)


@functools.cache
def _make_sc_gather():
    @functools.partial(
        pl.kernel,
        mesh=plsc.VectorSubcoreMesh(core_axis_name="c", subcore_axis_name="s"),
        out_type=jax.ShapeDtypeStruct((N_TOK, D), jnp.float32),
        scratch_types=[
            pltpu.VMEM((_B_PER_W,), jnp.int32),
            pltpu.VMEM((_B_PER_W, D), jnp.float32),
            pltpu.SemaphoreType.DMA,
        ],
    )
    def _sc_gather(idx_hbm, table_hbm, out_hbm, idx_v, rows_v, sem):
        wid = lax.axis_index("s") * _NC + lax.axis_index("c")
        base = wid * _B_PER_W
        pltpu.sync_copy(idx_hbm.at[pl.ds(base, _B_PER_W)], idx_v)
        pltpu.async_copy(table_hbm.at[idx_v], rows_v, sem).wait()
        pltpu.sync_copy(rows_v, out_hbm.at[pl.ds(base, _B_PER_W)])

    return _sc_gather


def kernel(inputs, beta, codebook):
    del beta
    b, c, h, w = inputs.shape
    x4 = jnp.transpose(inputs, (0, 2, 3, 1))
    x_flat = x4.reshape(-1, D)
    znorm = jnp.sum(x4 ** 2, axis=3).reshape(-1, 1)
    cnorm = jnp.sum(codebook ** 2, axis=1).reshape(1, -1)
    near2, perp = _nearest_and_perplexity(x_flat, codebook, znorm, cnorm)
    nearest = near2.reshape(-1)
    q_rows = _make_sc_gather()(nearest, codebook)
    quantized = jnp.transpose(q_rows.reshape(b, h, w, D), (0, 3, 1, 2))
    return quantized, perp[0, 0]


# speedup vs baseline: 1.0275x; 1.0275x over previous
import functools

import jax
import jax.numpy as jnp
from jax import lax
from jax.experimental import pallas as pl
from jax.experimental.pallas import tpu as pltpu
from jax.experimental.pallas import tpu_sc as plsc

N_TOK = 8192
K = 8192
D = 256
BT = 1024
N_BLOCKS = N_TOK // BT

_NC = 2
_NS = 16
_NW = _NC * _NS
_B_PER_W = N_TOK // _NW


def _argmin_body(x_ref, c_ref, zn_ref, cn_ref, near_ref, perp_ref, cnt_ref):
    i = pl.program_id(0)

    @pl.when(i == 0)
    def _init():
        cnt_ref[...] = jnp.zeros_like(cnt_ref)

    x = x_ref[...]
    xb = (2.0 * x).astype(jnp.bfloat16)
    dn = (((1,), (1,)), ((), ()))
    kio = lax.broadcasted_iota(jnp.int32, (BT, K // 2), 1)
    zn = zn_ref[...]
    conv0 = lax.dot_general(xb, c_ref[: K // 2], dn,
                            preferred_element_type=jnp.float32)
    dl = (zn - conv0) + cn_ref[:, : K // 2]
    m0 = jnp.min(dl, axis=1, keepdims=True)
    i0 = jnp.min(jnp.where(dl == m0, kio, K), axis=1, keepdims=True)
    conv1 = lax.dot_general(xb, c_ref[K // 2:], dn,
                            preferred_element_type=jnp.float32)
    dr = (zn - conv1) + cn_ref[:, K // 2:]
    m1 = jnp.min(dr, axis=1, keepdims=True)
    i1 = jnp.min(jnp.where(dr == m1, kio, K), axis=1, keepdims=True) + K // 2
    a = m0.astype(jnp.bfloat16).astype(jnp.float32)
    idx = jnp.where(a <= m1, i0, i1)
    near_ref[...] = idx

    hi = idx >> 7
    lo = idx & 127
    oh_hi = (hi == lax.broadcasted_iota(jnp.int32, (BT, 64), 1)
             ).astype(jnp.float32)
    oh_lo = (lo == lax.broadcasted_iota(jnp.int32, (BT, 128), 1)
             ).astype(jnp.float32)
    cnt_ref[...] += lax.dot_general(oh_hi, oh_lo, (((0,), (0,)), ((), ())),
                                    preferred_element_type=jnp.float32)

    @pl.when(i == N_BLOCKS - 1)
    def _fin():
        e = cnt_ref[...] * (1.0 / N_TOK)
        ent = jnp.sum(e * jnp.log(e + 1e-10))
        perp_ref[...] = jnp.broadcast_to(jnp.exp(-ent), (1, 1))


def _nearest_and_perplexity(x_flat, codebook, znorm, cnorm):
    return pl.pallas_call(
        _argmin_body,
        grid=(N_BLOCKS,),
        in_specs=[
            pl.BlockSpec((BT, D), lambda i: (i, 0)),
            pl.BlockSpec((K, D), lambda i: (0, 0)),
            pl.BlockSpec((BT, 1), lambda i: (i, 0)),
            pl.BlockSpec((1, K), lambda i: (0, 0)),
        ],
        out_specs=[
            pl.BlockSpec((BT, 1), lambda i: (i, 0)),
            pl.BlockSpec((1, 1), lambda i: (0, 0)),
        ],
        out_shape=[
            jax.ShapeDtypeStruct((N_TOK, 1), jnp.int32),
            jax.ShapeDtypeStruct((1, 1), jnp.float32),
        ],
        scratch_shapes=[
            pltpu.VMEM((64, 128), jnp.float32),
        ],
    )(x_flat, codebook, znorm, cnorm)


@functools.cache
def _make_sc_gather():
    @functools.partial(
        pl.kernel,
        mesh=plsc.VectorSubcoreMesh(core_axis_name="c", subcore_axis_name="s"),
        out_type=jax.ShapeDtypeStruct((N_TOK, D), jnp.float32),
        scratch_types=[
            pltpu.VMEM((_B_PER_W,), jnp.int32),
            pltpu.VMEM((_B_PER_W, D), jnp.float32),
            pltpu.SemaphoreType.DMA,
        ],
    )
    def _sc_gather(idx_hbm, table_hbm, out_hbm, idx_v, rows_v, sem):
        wid = lax.axis_index("s") * _NC + lax.axis_index("c")
        base = wid * _B_PER_W
        pltpu.sync_copy(idx_hbm.at[pl.ds(base, _B_PER_W)], idx_v)
        pltpu.async_copy(table_hbm.at[idx_v], rows_v, sem).wait()
        pltpu.sync_copy(rows_v, out_hbm.at[pl.ds(base, _B_PER_W)])

    return _sc_gather


def kernel(inputs, beta, codebook):
    del beta
    b, c, h, w = inputs.shape
    x4 = jnp.transpose(inputs, (0, 2, 3, 1))
    x_flat = x4.reshape(-1, D)
    znorm = jnp.sum(x4 ** 2, axis=3).reshape(-1, 1)
    cnorm = jnp.sum(codebook ** 2, axis=1).reshape(1, -1)
    near2, perp = _nearest_and_perplexity(x_flat, codebook, znorm, cnorm)
    nearest = near2.reshape(-1)
    q_rows = _make_sc_gather()(nearest, codebook)
    quantized = jnp.transpose(q_rows.reshape(b, h, w, D), (0, 3, 1, 2))
    return quantized, perp[0, 0]
